# baseline (speedup 1.0000x reference)
import functools

import jax
import jax.numpy as jnp
from jax import lax
from jax.experimental import pallas as pl
from jax.experimental.pallas import tpu as pltpu
from jax.experimental.pallas import tpu_sc as plsc

D = 256
K = 8192
B = 32
T = 576
N = B * T
COMMITMENT_WEIGHT = 0.25

M_BLK = 1024
ZONE_W = 2736
ZONE_PAD = 2816
CHUNK = 1408
GRID_M = (N // 2) // M_BLK

def _dist_argmin_body(zsq_ref, x_ref, et_ref, esq_ref, idx_ref, minv_ref):
    xm2 = (x_ref[...] * (-2.0)).astype(jnp.bfloat16)
    zsq = zsq_ref[...]
    run_min = jnp.full((M_BLK, 1), jnp.inf, dtype=jnp.float32)
    run_idx = jnp.zeros((M_BLK, 1), dtype=jnp.int32)
    lane = lax.broadcasted_iota(jnp.int32, (M_BLK, 128), 1)
    for zone in range(3):
        sl = pl.ds(zone * ZONE_PAD, ZONE_PAD)
        ab2 = jnp.dot(xm2, et_ref[:, sl], preferred_element_type=jnp.float32)
        runv = jnp.full((M_BLK, 128), jnp.inf, dtype=jnp.float32)
        runs = jnp.zeros((M_BLK, 128), dtype=jnp.int32)
        for sslab in range(ZONE_PAD // 128):
            co = sslab * 128
            ds = (zsq + ab2[:, co:co + 128]
                  + esq_ref[:, zone * ZONE_PAD + co:zone * ZONE_PAD + co + 128])
            lt = ds < runv
            runv = jnp.where(lt, ds, runv)
            runs = jnp.where(lt, jnp.int32(sslab), runs)
        zmin = jnp.min(runv, axis=1, keepdims=True)
        runc = runs * jnp.int32(128) + (lane + jnp.int32(zone * ZONE_W))
        cand = jnp.where(runv == zmin, runc, jnp.int32(2**30))
        zidx = jnp.min(cand, axis=1, keepdims=True)
        upd = zmin < run_min
        run_idx = jnp.where(upd, zidx, run_idx)
        run_min = jnp.where(upd, zmin, run_min)
        run_min = run_min.astype(jnp.bfloat16).astype(jnp.float32)
    idx_ref[...] = run_idx
    minv_ref[...] = run_min


def _vq_dist_argmin(xm2, emb_t, zsq, esq):
    return pl.pallas_call(
        _dist_argmin_body,
        grid=(GRID_M,),
        in_specs=[
            pl.BlockSpec((M_BLK, 1), lambda i: (i, 0)),
            pl.BlockSpec((M_BLK, D), lambda i: (i, 0)),
            pl.BlockSpec((D, 3 * ZONE_PAD), lambda i: (0, 0)),
            pl.BlockSpec((1, 3 * ZONE_PAD), lambda i: (0, 0)),
        ],
        out_specs=[
            pl.BlockSpec((M_BLK, 1), lambda i: (i, 0)),
            pl.BlockSpec((M_BLK, 1), lambda i: (i, 0)),
        ],
        out_shape=[
            jax.ShapeDtypeStruct((N // 2, 1), jnp.int32),
            jax.ShapeDtypeStruct((N // 2, 1), jnp.float32),
        ],
    )(zsq, xm2, emb_t, esq)


_NC = 2
_NS = 16
_NW = _NC * _NS
N_HALF = N // 2
_B_PER_W = N_HALF // _NW
_SC_CHUNK = 96
_SC_STEPS = _B_PER_W // _SC_CHUNK


@functools.cache
def _make_sc_gather():
    @functools.partial(
        pl.kernel,
        mesh=plsc.VectorSubcoreMesh(core_axis_name="c", subcore_axis_name="s"),
        out_type=jax.ShapeDtypeStruct((N_HALF, D), jnp.float32),
        scratch_types=[
            pltpu.VMEM((2, _SC_CHUNK), jnp.int32),
            pltpu.VMEM((2, _SC_CHUNK, D), jnp.float32),
            pltpu.SemaphoreType.DMA,
            pltpu.SemaphoreType.DMA,
            pltpu.SemaphoreType.DMA,
            pltpu.SemaphoreType.DMA,
        ],
    )
    def _sc_gather(idx_hbm, table_hbm, out_hbm, idx_v, rows_v, g0, g1, w0, w1):
        wid = lax.axis_index("s") * _NC + lax.axis_index("c")
        base = wid * _B_PER_W
        gsem = (g0, g1)
        wsem = (w0, w1)
        pltpu.sync_copy(idx_hbm.at[pl.ds(base, _SC_CHUNK)], idx_v.at[0])
        gath = {0: pltpu.async_copy(table_hbm.at[idx_v.at[0]], rows_v.at[0], g0)}
        wb = {}
        for c in range(_SC_STEPS):
            b = c % 2
            if c + 1 < _SC_STEPS:
                nb = (c + 1) % 2
                cb1 = base + (c + 1) * _SC_CHUNK
                if c - 1 >= 0:
                    wb[c - 1].wait()
                pltpu.sync_copy(idx_hbm.at[pl.ds(cb1, _SC_CHUNK)], idx_v.at[nb])
                gath[c + 1] = pltpu.async_copy(
                    table_hbm.at[idx_v.at[nb]], rows_v.at[nb], gsem[nb])
            gath[c].wait()
            wb[c] = pltpu.async_copy(
                rows_v.at[b], out_hbm.at[pl.ds(base + c * _SC_CHUNK, _SC_CHUNK)],
                wsem[b])
        wb[_SC_STEPS - 1].wait()
        wb[_SC_STEPS - 2].wait()

    return _sc_gather


def kernel(z_e, embeddings):
    input_shape = z_e.shape
    flat = z_e.reshape(-1, D)
    zsq = jnp.sum(flat * flat, axis=-1, keepdims=True)
    esq = jnp.sum(embeddings * embeddings, axis=-1)[None, :]
    et = embeddings.T.astype(jnp.bfloat16)
    et_pad = jnp.concatenate(
        [jnp.pad(et[:, z * ZONE_W:(z + 1) * ZONE_W],
                 ((0, 0), (0, ZONE_PAD - min(ZONE_W, K - z * ZONE_W))))
         for z in range(3)], axis=1)
    esq_pad = jnp.concatenate(
        [jnp.pad(esq[:, z * ZONE_W:(z + 1) * ZONE_W],
                 ((0, 0), (0, ZONE_PAD - min(ZONE_W, K - z * ZONE_W))),
                 constant_values=jnp.inf)
         for z in range(3)], axis=1)
    sc = _make_sc_gather()
    idx2a, minva = _vq_dist_argmin(flat[:N_HALF], et_pad, zsq[:N_HALF], esq_pad)
    idxa = idx2a.reshape(-1)
    zqa = sc(idxa, embeddings)
    idx2b, minvb = _vq_dist_argmin(flat[N_HALF:], et_pad, zsq[N_HALF:], esq_pad)
    idxb = idx2b.reshape(-1)
    zqb = sc(idxb, embeddings)
    idx = jnp.concatenate([idxa, idxb])
    z_q = jnp.concatenate([zqa, zqb], axis=0)
    commit_loss = (jnp.sum(minva) + jnp.sum(minvb)) * (COMMITMENT_WEIGHT / (N * D))
    z_q = z_q.reshape(input_shape)
    indices_out = idx.reshape(input_shape[:-1])
    return (z_q, indices_out, commit_loss, z_q)


# speedup vs baseline: 1.0933x; 1.0933x over previous
import functools

import jax
import jax.numpy as jnp
from jax import lax
from jax.experimental import pallas as pl
from jax.experimental.pallas import tpu as pltpu
from jax.experimental.pallas import tpu_sc as plsc

D = 256
K = 8192
B = 32
T = 576
N = B * T
COMMITMENT_WEIGHT = 0.25

M_BLK = 1024
ZONE_W = 2736
ZONE_PAD = 2816
CHUNK = 1408
GRID_M = N // M_BLK

def _dist_argmin_body(zsq_ref, x_ref, et_ref, esq_ref, idx_ref, minv_ref):
    xm2 = (x_ref[...] * (-2.0)).astype(jnp.bfloat16)
    zsq = zsq_ref[...]
    run_min = jnp.full((M_BLK, 1), jnp.inf, dtype=jnp.float32)
    run_idx = jnp.zeros((M_BLK, 1), dtype=jnp.int32)
    lane = lax.broadcasted_iota(jnp.int32, (M_BLK, 128), 1)
    for zone in range(3):
        sl = pl.ds(zone * ZONE_PAD, ZONE_PAD)
        ab2 = lax.dot_general(xm2, et_ref[sl, :], (((1,), (1,)), ((), ())),
                              preferred_element_type=jnp.float32)
        runv = jnp.full((M_BLK, 128), jnp.inf, dtype=jnp.float32)
        runs = jnp.zeros((M_BLK, 128), dtype=jnp.int32)
        for sslab in range(ZONE_PAD // 128):
            co = sslab * 128
            ds = (zsq + ab2[:, co:co + 128]
                  + esq_ref[:, zone * ZONE_PAD + co:zone * ZONE_PAD + co + 128])
            lt = ds < runv
            runv = jnp.where(lt, ds, runv)
            runs = jnp.where(lt, jnp.int32(sslab), runs)
        zmin = jnp.min(runv, axis=1, keepdims=True)
        runc = runs * jnp.int32(128) + (lane + jnp.int32(zone * ZONE_W))
        cand = jnp.where(runv == zmin, runc, jnp.int32(2**30))
        zidx = jnp.min(cand, axis=1, keepdims=True)
        upd = zmin < run_min
        run_idx = jnp.where(upd, zidx, run_idx)
        run_min = jnp.where(upd, zmin, run_min)
        run_min = run_min.astype(jnp.bfloat16).astype(jnp.float32)
    idx_ref[...] = run_idx
    minv_ref[...] = run_min


def _vq_dist_argmin(xm2, emb_t, zsq, esq):
    return pl.pallas_call(
        _dist_argmin_body,
        grid=(GRID_M,),
        in_specs=[
            pl.BlockSpec((M_BLK, 1), lambda i: (i, 0)),
            pl.BlockSpec((M_BLK, D), lambda i: (i, 0)),
            pl.BlockSpec((3 * ZONE_PAD, D), lambda i: (0, 0)),
            pl.BlockSpec((1, 3 * ZONE_PAD), lambda i: (0, 0)),
        ],
        out_specs=[
            pl.BlockSpec((M_BLK, 1), lambda i: (i, 0)),
            pl.BlockSpec((M_BLK, 1), lambda i: (i, 0)),
        ],
        out_shape=[
            jax.ShapeDtypeStruct((N, 1), jnp.int32),
            jax.ShapeDtypeStruct((N, 1), jnp.float32),
        ],
    )(zsq, xm2, emb_t, esq)


_NC = 2
_NS = 16
_NW = _NC * _NS
_B_PER_W = N // _NW
_SC_CHUNK = 96
_SC_STEPS = _B_PER_W // _SC_CHUNK


@functools.cache
def _make_sc_gather():
    @functools.partial(
        pl.kernel,
        mesh=plsc.VectorSubcoreMesh(core_axis_name="c", subcore_axis_name="s"),
        out_type=jax.ShapeDtypeStruct((N, D), jnp.float32),
        scratch_types=[
            pltpu.VMEM((2, _SC_CHUNK), jnp.int32),
            pltpu.VMEM((2, _SC_CHUNK, D), jnp.float32),
            pltpu.SemaphoreType.DMA,
            pltpu.SemaphoreType.DMA,
            pltpu.SemaphoreType.DMA,
            pltpu.SemaphoreType.DMA,
        ],
    )
    def _sc_gather(idx_hbm, table_hbm, out_hbm, idx_v, rows_v, g0, g1, w0, w1):
        wid = lax.axis_index("s") * _NC + lax.axis_index("c")
        base = wid * _B_PER_W
        gsem = (g0, g1)
        wsem = (w0, w1)
        pltpu.sync_copy(idx_hbm.at[pl.ds(base, _SC_CHUNK)], idx_v.at[0])
        gath = {0: pltpu.async_copy(table_hbm.at[idx_v.at[0]], rows_v.at[0], g0)}
        wb = {}
        for c in range(_SC_STEPS):
            b = c % 2
            if c + 1 < _SC_STEPS:
                nb = (c + 1) % 2
                cb1 = base + (c + 1) * _SC_CHUNK
                if c - 1 >= 0:
                    wb[c - 1].wait()
                pltpu.sync_copy(idx_hbm.at[pl.ds(cb1, _SC_CHUNK)], idx_v.at[nb])
                gath[c + 1] = pltpu.async_copy(
                    table_hbm.at[idx_v.at[nb]], rows_v.at[nb], gsem[nb])
            gath[c].wait()
            wb[c] = pltpu.async_copy(
                rows_v.at[b], out_hbm.at[pl.ds(base + c * _SC_CHUNK, _SC_CHUNK)],
                wsem[b])
        wb[_SC_STEPS - 1].wait()
        wb[_SC_STEPS - 2].wait()

    return _sc_gather


def kernel(z_e, embeddings):
    input_shape = z_e.shape
    flat = z_e.reshape(-1, D)
    zsq = jnp.sum(flat * flat, axis=-1, keepdims=True)
    esq = jnp.sum(embeddings * embeddings, axis=-1)[None, :]
    et = embeddings.astype(jnp.bfloat16)
    et_pad = jnp.concatenate(
        [jnp.pad(et[z * ZONE_W:(z + 1) * ZONE_W, :],
                 ((0, ZONE_PAD - min(ZONE_W, K - z * ZONE_W)), (0, 0)))
         for z in range(3)], axis=0)
    esq_pad = jnp.concatenate(
        [jnp.pad(esq[:, z * ZONE_W:(z + 1) * ZONE_W],
                 ((0, 0), (0, ZONE_PAD - min(ZONE_W, K - z * ZONE_W))),
                 constant_values=jnp.inf)
         for z in range(3)], axis=1)
    idx2, minv = _vq_dist_argmin(flat, et_pad, zsq, esq_pad)
    idx = idx2.reshape(-1)
    z_q = _make_sc_gather()(idx, embeddings)
    commit_loss = jnp.sum(minv) * (COMMITMENT_WEIGHT / (N * D))
    z_q = z_q.reshape(input_shape)
    indices_out = idx.reshape(input_shape[:-1])
    return (z_q, indices_out, commit_loss, z_q)


# speedup vs baseline: 1.1337x; 1.0369x over previous
import functools

import jax
import jax.numpy as jnp
from jax import lax
from jax.experimental import pallas as pl
from jax.experimental.pallas import tpu as pltpu
from jax.experimental.pallas import tpu_sc as plsc

D = 256
K = 8192
B = 32
T = 576
N = B * T
COMMITMENT_WEIGHT = 0.25

M_BLK = 1024
ZONE_W = 2736
ZONE_PAD = 2816
CHUNK = 1408
GRID_M = N // M_BLK

def _dist_argmin_body(zsq_ref, x_ref, et_ref, esq_ref, idx_ref, minv_ref):
    xm2 = (x_ref[...] * (-2.0)).astype(jnp.bfloat16)
    zsq = zsq_ref[...]
    run_min = jnp.full((M_BLK, 1), jnp.inf, dtype=jnp.float32)
    run_idx = jnp.zeros((M_BLK, 1), dtype=jnp.int32)
    lane = lax.broadcasted_iota(jnp.int32, (M_BLK, 128), 1)
    for zone in range(3):
        sl = pl.ds(zone * ZONE_PAD, ZONE_PAD)
        ab2 = jnp.dot(xm2, et_ref[:, sl], preferred_element_type=jnp.float32)
        runv = jnp.full((M_BLK, 128), jnp.inf, dtype=jnp.float32)
        runs = jnp.zeros((M_BLK, 128), dtype=jnp.int32)
        for sslab in range(ZONE_PAD // 128):
            co = sslab * 128
            ds = (zsq + ab2[:, co:co + 128]
                  + esq_ref[:, zone * ZONE_PAD + co:zone * ZONE_PAD + co + 128])
            lt = ds < runv
            runv = jnp.where(lt, ds, runv)
            runs = jnp.where(lt, jnp.int32(sslab), runs)
        zmin = jnp.min(runv, axis=1, keepdims=True)
        runc = runs * jnp.int32(128) + (lane + jnp.int32(zone * ZONE_W))
        cand = jnp.where(runv == zmin, runc, jnp.int32(2**30))
        zidx = jnp.min(cand, axis=1, keepdims=True)
        upd = zmin < run_min
        run_idx = jnp.where(upd, zidx, run_idx)
        run_min = jnp.where(upd, zmin, run_min)
        run_min = run_min.astype(jnp.bfloat16).astype(jnp.float32)
    idx_ref[...] = run_idx
    minv_ref[...] = run_min


def _vq_dist_argmin(xm2, emb_t, zsq, esq):
    return pl.pallas_call(
        _dist_argmin_body,
        grid=(GRID_M,),
        in_specs=[
            pl.BlockSpec((M_BLK, 1), lambda i: (i, 0)),
            pl.BlockSpec((M_BLK, D), lambda i: (i, 0)),
            pl.BlockSpec((D, 3 * ZONE_PAD), lambda i: (0, 0)),
            pl.BlockSpec((1, 3 * ZONE_PAD), lambda i: (0, 0)),
        ],
        out_specs=[
            pl.BlockSpec((M_BLK, 1), lambda i: (i, 0)),
            pl.BlockSpec((M_BLK, 1), lambda i: (i, 0)),
        ],
        out_shape=[
            jax.ShapeDtypeStruct((N, 1), jnp.int32),
            jax.ShapeDtypeStruct((N, 1), jnp.float32),
        ],
    )(zsq, xm2, emb_t, esq)


_NC = 2
_NS = 16
_NW = _NC * _NS
_B_PER_W = N // _NW
_SC_CHUNK = 96
_SC_STEPS = _B_PER_W // _SC_CHUNK


@functools.cache
def _make_sc_gather():
    @functools.partial(
        pl.kernel,
        mesh=plsc.VectorSubcoreMesh(core_axis_name="c", subcore_axis_name="s"),
        out_type=jax.ShapeDtypeStruct((N, D), jnp.float32),
        scratch_types=[
            pltpu.VMEM((2, _SC_CHUNK), jnp.int32),
            pltpu.VMEM((2, _SC_CHUNK, D), jnp.float32),
            pltpu.SemaphoreType.DMA,
            pltpu.SemaphoreType.DMA,
            pltpu.SemaphoreType.DMA,
            pltpu.SemaphoreType.DMA,
        ],
    )
    def _sc_gather(idx_hbm, table_hbm, out_hbm, idx_v, rows_v, g0, g1, w0, w1):
        wid = lax.axis_index("s") * _NC + lax.axis_index("c")
        base = wid * _B_PER_W
        gsem = (g0, g1)
        wsem = (w0, w1)
        pltpu.sync_copy(idx_hbm.at[pl.ds(base, _SC_CHUNK)], idx_v.at[0])
        gath = {0: pltpu.async_copy(table_hbm.at[idx_v.at[0]], rows_v.at[0], g0)}
        wb = {}
        for c in range(_SC_STEPS):
            b = c % 2
            if c + 1 < _SC_STEPS:
                nb = (c + 1) % 2
                cb1 = base + (c + 1) * _SC_CHUNK
                if c - 1 >= 0:
                    wb[c - 1].wait()
                pltpu.sync_copy(idx_hbm.at[pl.ds(cb1, _SC_CHUNK)], idx_v.at[nb])
                gath[c + 1] = pltpu.async_copy(
                    table_hbm.at[idx_v.at[nb]], rows_v.at[nb], gsem[nb])
            gath[c].wait()
            wb[c] = pltpu.async_copy(
                rows_v.at[b], out_hbm.at[pl.ds(base + c * _SC_CHUNK, _SC_CHUNK)],
                wsem[b])
        wb[_SC_STEPS - 1].wait()
        wb[_SC_STEPS - 2].wait()

    return _sc_gather


def kernel(z_e, embeddings):
    input_shape = z_e.shape
    flat = z_e.reshape(-1, D)
    zsq = jnp.sum(flat * flat, axis=-1, keepdims=True)
    esq = jnp.sum(embeddings * embeddings, axis=-1)[None, :]
    et = embeddings.T.astype(jnp.bfloat16)
    et_pad = jnp.concatenate(
        [jnp.pad(et[:, z * ZONE_W:(z + 1) * ZONE_W],
                 ((0, 0), (0, ZONE_PAD - min(ZONE_W, K - z * ZONE_W))))
         for z in range(3)], axis=1)
    esq_pad = jnp.concatenate(
        [jnp.pad(esq[:, z * ZONE_W:(z + 1) * ZONE_W],
                 ((0, 0), (0, ZONE_PAD - min(ZONE_W, K - z * ZONE_W))),
                 constant_values=jnp.inf)
         for z in range(3)], axis=1)
    idx2, minv = _vq_dist_argmin(flat, et_pad, zsq, esq_pad)
    idx = idx2.reshape(-1)
    z_q = _make_sc_gather()(idx, embeddings)
    commit_loss = jnp.sum(minv) * (COMMITMENT_WEIGHT / (N * D))
    z_q = z_q.reshape(input_shape)
    indices_out = idx.reshape(input_shape[:-1])
    return (z_q, indices_out, commit_loss, z_q)
